# baseline (speedup 1.0000x reference)
import functools

import jax
import jax.numpy as jnp
from jax import lax
from jax.experimental import pallas as pl
from jax.experimental.pallas import tpu as pltpu
from jax.experimental.pallas import tpu_sc as plsc

B = 64
S = 1024
D = 768
L = 16

G = 4
BG = B // G
NG = BG * S

NUM_WORKERS = 32
PPW = S // NUM_WORKERS
PSTEP = 4
RSTEP = PSTEP * BG
NSTEPS = PPW // PSTEP


def _store_out_idx(out_idx_v, buf, wbase, t):
    for m in range(PSTEP):
        vec = lax.iota(jnp.int32, L) * S + (wbase + t * PSTEP + m)
        out_idx_v[buf, pl.ds(m * L, L)] = vec


def _gather_body(xt_hbm, tok_hbm, out_hbm, idx_v, rows_v, out_idx_v,
                 g0, g1, o0, o1):
    wid = lax.axis_index("s") * 2 + lax.axis_index("c")
    wbase = wid * PPW
    gsem = (g0, g1)
    osem = (o0, o1)

    pltpu.sync_copy(xt_hbm.at[pl.ds(wbase * BG, PPW * BG)], idx_v)

    def issue_gather(t, buf):
        pltpu.async_copy(tok_hbm.at[idx_v.at[pl.ds(t * RSTEP, RSTEP)]],
                         rows_v.at[buf], gsem[buf])

    def wait_gather(buf):
        pltpu.make_async_copy(tok_hbm.at[idx_v.at[pl.ds(0, RSTEP)]],
                              rows_v.at[buf], gsem[buf]).wait()

    def issue_store(buf):
        pltpu.async_copy(rows_v.at[buf], out_hbm.at[out_idx_v.at[buf]],
                         osem[buf])

    def wait_store(buf):
        pltpu.make_async_copy(rows_v.at[buf], out_hbm.at[out_idx_v.at[buf]],
                              osem[buf]).wait()

    issue_gather(0, 0)
    issue_gather(1, 1)
    wait_gather(0)
    _store_out_idx(out_idx_v, 0, wbase, 0)
    issue_store(0)

    def pair(tt, carry):
        t = 2 * tt + 1
        wait_gather(1)
        wait_store(0)
        issue_gather(t + 1, 0)
        _store_out_idx(out_idx_v, 1, wbase, t)
        issue_store(1)

        wait_gather(0)
        wait_store(1)
        issue_gather(t + 2, 1)
        _store_out_idx(out_idx_v, 0, wbase, t + 1)
        issue_store(0)
        return carry

    lax.fori_loop(0, (NSTEPS - 2) // 2, pair, 0)

    wait_gather(1)
    wait_store(0)
    _store_out_idx(out_idx_v, 1, wbase, NSTEPS - 1)
    issue_store(1)
    wait_store(1)


def _sc_gather(xt_g, token_emb):
    mesh = plsc.VectorSubcoreMesh(core_axis_name="c", subcore_axis_name="s")
    f = functools.partial(
        pl.kernel,
        out_type=jax.ShapeDtypeStruct((NG, D), jnp.float32),
        mesh=mesh,
        scratch_types=[
            pltpu.VMEM((PPW * BG,), jnp.int32),
            pltpu.VMEM((2, RSTEP, D), jnp.float32),
            pltpu.VMEM((2, RSTEP), jnp.int32),
            pltpu.SemaphoreType.DMA,
            pltpu.SemaphoreType.DMA,
            pltpu.SemaphoreType.DMA,
            pltpu.SemaphoreType.DMA,
        ],
    )(_gather_body)
    return f(xt_g, token_emb)


def _add_body(tok_ref, pos_ref, out_ref):
    out_ref[...] = tok_ref[...] + pos_ref[...]


def _tc_add(tok_g, pos2d):
    return pl.pallas_call(
        _add_body,
        grid=(BG,),
        in_specs=[
            pl.BlockSpec((1, S, D), lambda b: (b, 0, 0)),
            pl.BlockSpec((S, D), lambda b: (0, 0)),
        ],
        out_specs=pl.BlockSpec((1, S, D), lambda b: (b, 0, 0)),
        out_shape=jax.ShapeDtypeStruct((BG, S, D), jnp.float32),
    )(tok_g, pos2d)


@jax.jit
def _emb(x, token_emb, pos2d):
    outs = []
    for g in range(G):
        xg = x[g * BG:(g + 1) * BG]
        xt = xg.T.reshape(NG)
        tok = _sc_gather(xt, token_emb)
        outs.append(tok.reshape(BG, S, D) + pos2d[None])
    return jnp.concatenate(outs, axis=0)


def kernel(x, token_emb, pos_emb):
    pos2d = pos_emb.reshape(S, D)
    return _emb(x, token_emb, pos2d)
